# baseline (speedup 1.0000x reference)
import functools

import jax
import jax.numpy as jnp
from jax import lax
from jax.experimental import pallas as pl
from jax.experimental.pallas import tpu as pltpu
from jax.experimental.pallas import tpu_sc as plsc

N = 1048576
VOL = 256
VOL3 = VOL * VOL * VOL
NW = 32
SPW = N // NW
C = 1024
G = C // 16
NCHUNK = SPW // C


def _tec_body(smp_hbm, table_hbm, out_hbm,
              smp0, smp1, idx0, idx1, val0, val1,
              wx0, wx1, wy0, wy1, wz0, wz1, out0, out1, semA, semB):
    wid = lax.axis_index("s") * 2 + lax.axis_index("c")
    bufs = ((smp0, idx0, val0, wx0, wy0, wz0, out0, semA),
            (smp1, idx1, val1, wx1, wy1, wz1, out1, semB))

    def stage(k, b):
        smp_v = bufs[b][0]
        base = wid * SPW + k * C
        pltpu.sync_copy(smp_hbm.at[pl.ds(0, 7), pl.ds(base, C)], smp_v)

    def pass1(b):
        smp_v, idx_v, _, wx_v, wy_v, wz_v, _, _ = bufs[b]

        def idx_body(g, carry2):
            r = g * 16
            c0 = smp_v[0, pl.ds(r, 16)]
            c1 = smp_v[1, pl.ds(r, 16)]
            c2 = smp_v[2, pl.ds(r, 16)]
            c3 = smp_v[3, pl.ds(r, 16)]
            c4 = smp_v[4, pl.ds(r, 16)]
            c5 = smp_v[5, pl.ds(r, 16)]
            fl = smp_v[6, pl.ds(r, 16)]
            is_yin = fl == 0.0
            x = (jnp.where(is_yin, c0, c3) + 1.0) * 127.5
            y = (jnp.where(is_yin, c1, c4) + 1.0) * 127.5
            z = (jnp.where(is_yin, c2, c5) + 1.0) * 127.5
            xi = x.astype(jnp.int32)
            yi = y.astype(jnp.int32)
            zi = z.astype(jnp.int32)
            wx_v[pl.ds(r, 16)] = x - xi.astype(jnp.float32)
            wy_v[pl.ds(r, 16)] = y - yi.astype(jnp.float32)
            wz_v[pl.ds(r, 16)] = z - zi.astype(jnp.float32)
            xt0 = xi + (xi & 0x80) * 7
            xj = xi + 1
            xt1 = xj + (xj & 0x80) * 7
            yt0 = (yi << 7) + ((yi >> 3) << 10)
            yj = yi + 1
            yt1 = (yj << 7) + ((yj >> 3) << 10)
            vz0 = jnp.where(is_yin, 0, VOL3) + (zi << 16)
            zy00 = vz0 + yt0
            zy01 = vz0 + yt1
            zy10 = zy00 + 65536
            zy11 = zy01 + 65536
            idx_v[pl.ds(0 * C + r, 16)] = zy00 + xt0
            idx_v[pl.ds(1 * C + r, 16)] = zy00 + xt1
            idx_v[pl.ds(2 * C + r, 16)] = zy01 + xt0
            idx_v[pl.ds(3 * C + r, 16)] = zy01 + xt1
            idx_v[pl.ds(4 * C + r, 16)] = zy10 + xt0
            idx_v[pl.ds(5 * C + r, 16)] = zy10 + xt1
            idx_v[pl.ds(6 * C + r, 16)] = zy11 + xt0
            idx_v[pl.ds(7 * C + r, 16)] = zy11 + xt1
            return carry2

        lax.fori_loop(0, G, idx_body, 0, unroll=False)

    def fire(b):
        _, idx_v, val_v, _, _, _, _, sem = bufs[b]
        pltpu.async_copy(table_hbm.at[idx_v], val_v, sem)

    def drain(b):
        _, idx_v, val_v, _, _, _, _, sem = bufs[b]
        pltpu.make_async_copy(table_hbm.at[idx_v], val_v, sem).wait()

    def pass2(k, b):
        _, _, val_v, wx_v, wy_v, wz_v, out_v, _ = bufs[b]

        def mix_body(g, carry2):
            r = g * 16
            wx = wx_v[pl.ds(r, 16)]
            wy = wy_v[pl.ds(r, 16)]
            wz = wz_v[pl.ds(r, 16)]
            v000 = val_v[pl.ds(0 * C + r, 16)]
            v001 = val_v[pl.ds(1 * C + r, 16)]
            v010 = val_v[pl.ds(2 * C + r, 16)]
            v011 = val_v[pl.ds(3 * C + r, 16)]
            v100 = val_v[pl.ds(4 * C + r, 16)]
            v101 = val_v[pl.ds(5 * C + r, 16)]
            v110 = val_v[pl.ds(6 * C + r, 16)]
            v111 = val_v[pl.ds(7 * C + r, 16)]
            a00 = v000 + wx * (v001 - v000)
            a01 = v010 + wx * (v011 - v010)
            a10 = v100 + wx * (v101 - v100)
            a11 = v110 + wx * (v111 - v110)
            b0 = a00 + wy * (a01 - a00)
            b1 = a10 + wy * (a11 - a10)
            out_v[pl.ds(r, 16)] = b0 + wz * (b1 - b0)
            return carry2

        lax.fori_loop(0, G, mix_body, 0, unroll=False)
        base = wid * SPW + k * C
        pltpu.sync_copy(out_v, out_hbm.at[pl.ds(base, C)])

    stage(0, 0)
    pass1(0)
    fire(0)

    def pipe_body(k2, carry):
        k1 = 2 * k2 + 1
        stage(k1, 1)
        pass1(1)
        fire(1)
        drain(0)
        pass2(2 * k2, 0)
        kn = 2 * k2 + 2

        @pl.when(kn < NCHUNK)
        def _():
            stage(kn, 0)
            pass1(0)
            fire(0)

        drain(1)
        pass2(k1, 1)
        return carry

    lax.fori_loop(0, NCHUNK // 2, pipe_body, 0, unroll=False)


@jax.jit
def _run(samples_t, table):
    mesh = plsc.VectorSubcoreMesh(core_axis_name="c", subcore_axis_name="s")
    f = functools.partial(
        pl.kernel,
        mesh=mesh,
        compiler_params=pltpu.CompilerParams(use_tc_tiling_on_sc=True,
                                             needs_layout_passes=False),
        out_type=jax.ShapeDtypeStruct((N,), jnp.float32),
        scratch_types=[
            pltpu.VMEM((7, C), jnp.float32),
            pltpu.VMEM((7, C), jnp.float32),
            pltpu.VMEM((8 * C,), jnp.int32),
            pltpu.VMEM((8 * C,), jnp.int32),
            pltpu.VMEM((8 * C,), jnp.float32),
            pltpu.VMEM((8 * C,), jnp.float32),
            pltpu.VMEM((C,), jnp.float32),
            pltpu.VMEM((C,), jnp.float32),
            pltpu.VMEM((C,), jnp.float32),
            pltpu.VMEM((C,), jnp.float32),
            pltpu.VMEM((C,), jnp.float32),
            pltpu.VMEM((C,), jnp.float32),
            pltpu.VMEM((C,), jnp.float32),
            pltpu.VMEM((C,), jnp.float32),
            pltpu.SemaphoreType.DMA,
            pltpu.SemaphoreType.DMA,
        ],
    )(_tec_body)
    return f(samples_t, table)


def _tile_view(vol):
    v5 = vol.reshape(VOL, VOL // 8, 8, 2, 128)
    return v5.transpose(0, 1, 3, 2, 4).reshape(-1)


def kernel(norm_samples, alpha_volume_yin, alpha_volume_yang):
    table = jnp.concatenate(
        [_tile_view(alpha_volume_yin), _tile_view(alpha_volume_yang)])
    return _run(norm_samples.T, table)


# speedup vs baseline: 1.0349x; 1.0349x over previous
import functools

import jax
import jax.numpy as jnp
from jax import lax
from jax.experimental import pallas as pl
from jax.experimental.pallas import tpu as pltpu
from jax.experimental.pallas import tpu_sc as plsc

N = 1048576
VOL = 256
VOL3 = VOL * VOL * VOL
NW = 32
SPW = N // NW
C = 1024
G = C // 16
NCHUNK = SPW // C
DEPTH = 4


def _tec_body(smp_hbm, table_hbm, out_hbm, *scr):
    wid = lax.axis_index("s") * 2 + lax.axis_index("c")
    bufs = tuple(scr[8 * b:8 * b + 8] for b in range(DEPTH))

    def stage(k, b):
        smp_v = bufs[b][0]
        base = wid * SPW + k * C
        pltpu.sync_copy(smp_hbm.at[pl.ds(0, 7), pl.ds(base, C)], smp_v)

    def pass1(b):
        smp_v, idx_v, _, wx_v, wy_v, wz_v, _, _ = bufs[b]

        def idx_body(g, carry2):
            r = g * 16
            c0 = smp_v[0, pl.ds(r, 16)]
            c1 = smp_v[1, pl.ds(r, 16)]
            c2 = smp_v[2, pl.ds(r, 16)]
            c3 = smp_v[3, pl.ds(r, 16)]
            c4 = smp_v[4, pl.ds(r, 16)]
            c5 = smp_v[5, pl.ds(r, 16)]
            fl = smp_v[6, pl.ds(r, 16)]
            is_yin = fl == 0.0
            x = (jnp.where(is_yin, c0, c3) + 1.0) * 127.5
            y = (jnp.where(is_yin, c1, c4) + 1.0) * 127.5
            z = (jnp.where(is_yin, c2, c5) + 1.0) * 127.5
            xi = x.astype(jnp.int32)
            yi = y.astype(jnp.int32)
            zi = z.astype(jnp.int32)
            wx_v[pl.ds(r, 16)] = x - xi.astype(jnp.float32)
            wy_v[pl.ds(r, 16)] = y - yi.astype(jnp.float32)
            wz_v[pl.ds(r, 16)] = z - zi.astype(jnp.float32)
            xt0 = xi + (xi & 0x80) * 7
            xj = xi + 1
            xt1 = xj + (xj & 0x80) * 7
            yt0 = (yi << 7) + ((yi >> 3) << 10)
            yj = yi + 1
            yt1 = (yj << 7) + ((yj >> 3) << 10)
            vz0 = jnp.where(is_yin, 0, VOL3) + (zi << 16)
            zy00 = vz0 + yt0
            zy01 = vz0 + yt1
            zy10 = zy00 + 65536
            zy11 = zy01 + 65536
            idx_v[pl.ds(0 * C + r, 16)] = zy00 + xt0
            idx_v[pl.ds(1 * C + r, 16)] = zy00 + xt1
            idx_v[pl.ds(2 * C + r, 16)] = zy01 + xt0
            idx_v[pl.ds(3 * C + r, 16)] = zy01 + xt1
            idx_v[pl.ds(4 * C + r, 16)] = zy10 + xt0
            idx_v[pl.ds(5 * C + r, 16)] = zy10 + xt1
            idx_v[pl.ds(6 * C + r, 16)] = zy11 + xt0
            idx_v[pl.ds(7 * C + r, 16)] = zy11 + xt1
            return carry2

        lax.fori_loop(0, G, idx_body, 0, unroll=False)

    def fire(b):
        _, idx_v, val_v, _, _, _, _, sem = bufs[b]
        pltpu.async_copy(table_hbm.at[idx_v], val_v, sem)

    def drain(b):
        _, idx_v, val_v, _, _, _, _, sem = bufs[b]
        pltpu.make_async_copy(table_hbm.at[idx_v], val_v, sem).wait()

    def pass2(k, b):
        _, _, val_v, wx_v, wy_v, wz_v, out_v, _ = bufs[b]

        def mix_body(g, carry2):
            r = g * 16
            wx = wx_v[pl.ds(r, 16)]
            wy = wy_v[pl.ds(r, 16)]
            wz = wz_v[pl.ds(r, 16)]
            v000 = val_v[pl.ds(0 * C + r, 16)]
            v001 = val_v[pl.ds(1 * C + r, 16)]
            v010 = val_v[pl.ds(2 * C + r, 16)]
            v011 = val_v[pl.ds(3 * C + r, 16)]
            v100 = val_v[pl.ds(4 * C + r, 16)]
            v101 = val_v[pl.ds(5 * C + r, 16)]
            v110 = val_v[pl.ds(6 * C + r, 16)]
            v111 = val_v[pl.ds(7 * C + r, 16)]
            a00 = v000 + wx * (v001 - v000)
            a01 = v010 + wx * (v011 - v010)
            a10 = v100 + wx * (v101 - v100)
            a11 = v110 + wx * (v111 - v110)
            b0 = a00 + wy * (a01 - a00)
            b1 = a10 + wy * (a11 - a10)
            out_v[pl.ds(r, 16)] = b0 + wz * (b1 - b0)
            return carry2

        lax.fori_loop(0, G, mix_body, 0, unroll=False)
        base = wid * SPW + k * C
        pltpu.sync_copy(out_v, out_hbm.at[pl.ds(base, C)])

    for b in range(DEPTH - 1):
        stage(b, b)
        pass1(b)
        fire(b)

    def pipe_body(k2, carry):
        for b in range(DEPTH):
            k = k2 * DEPTH + b
            kf = k + DEPTH - 1
            bf = (b + DEPTH - 1) % DEPTH

            @pl.when(kf < NCHUNK)
            def _(kf=kf, bf=bf):
                stage(kf, bf)
                pass1(bf)
                fire(bf)

            drain(b)
            pass2(k, b)
        return carry

    lax.fori_loop(0, NCHUNK // DEPTH, pipe_body, 0, unroll=False)


@jax.jit
def _run(samples_t, table):
    mesh = plsc.VectorSubcoreMesh(core_axis_name="c", subcore_axis_name="s")
    f = functools.partial(
        pl.kernel,
        mesh=mesh,
        compiler_params=pltpu.CompilerParams(use_tc_tiling_on_sc=True,
                                             needs_layout_passes=False),
        out_type=jax.ShapeDtypeStruct((N,), jnp.float32),
        scratch_types=[t for _ in range(DEPTH) for t in (
            pltpu.VMEM((7, C), jnp.float32),
            pltpu.VMEM((8 * C,), jnp.int32),
            pltpu.VMEM((8 * C,), jnp.float32),
            pltpu.VMEM((C,), jnp.float32),
            pltpu.VMEM((C,), jnp.float32),
            pltpu.VMEM((C,), jnp.float32),
            pltpu.VMEM((C,), jnp.float32),
            pltpu.SemaphoreType.DMA,
        )],
    )(_tec_body)
    return f(samples_t, table)


def _tile_view(vol):
    v5 = vol.reshape(VOL, VOL // 8, 8, 2, 128)
    return v5.transpose(0, 1, 3, 2, 4).reshape(-1)


def kernel(norm_samples, alpha_volume_yin, alpha_volume_yang):
    table = jnp.concatenate(
        [_tile_view(alpha_volume_yin), _tile_view(alpha_volume_yang)])
    return _run(norm_samples.T, table)


# speedup vs baseline: 1.0369x; 1.0019x over previous
import functools

import jax
import jax.numpy as jnp
from jax import lax
from jax.experimental import pallas as pl
from jax.experimental.pallas import tpu as pltpu
from jax.experimental.pallas import tpu_sc as plsc

N = 1048576
VOL = 256
VOL3 = VOL * VOL * VOL
NW = 32
SPW = N // NW
C = 512
G = C // 16
NCHUNK = SPW // C
DEPTH = 8


def _tec_body(smp_hbm, table_hbm, out_hbm, *scr):
    wid = lax.axis_index("s") * 2 + lax.axis_index("c")
    bufs = tuple(scr[8 * b:8 * b + 8] for b in range(DEPTH))

    def stage(k, b):
        smp_v = bufs[b][0]
        base = wid * SPW + k * C
        pltpu.sync_copy(smp_hbm.at[pl.ds(0, 7), pl.ds(base, C)], smp_v)

    def pass1(b):
        smp_v, idx_v, _, wx_v, wy_v, wz_v, _, _ = bufs[b]

        def idx_body(g, carry2):
            r = g * 16
            c0 = smp_v[0, pl.ds(r, 16)]
            c1 = smp_v[1, pl.ds(r, 16)]
            c2 = smp_v[2, pl.ds(r, 16)]
            c3 = smp_v[3, pl.ds(r, 16)]
            c4 = smp_v[4, pl.ds(r, 16)]
            c5 = smp_v[5, pl.ds(r, 16)]
            fl = smp_v[6, pl.ds(r, 16)]
            is_yin = fl == 0.0
            x = (jnp.where(is_yin, c0, c3) + 1.0) * 127.5
            y = (jnp.where(is_yin, c1, c4) + 1.0) * 127.5
            z = (jnp.where(is_yin, c2, c5) + 1.0) * 127.5
            xi = x.astype(jnp.int32)
            yi = y.astype(jnp.int32)
            zi = z.astype(jnp.int32)
            wx_v[pl.ds(r, 16)] = x - xi.astype(jnp.float32)
            wy_v[pl.ds(r, 16)] = y - yi.astype(jnp.float32)
            wz_v[pl.ds(r, 16)] = z - zi.astype(jnp.float32)
            xt0 = xi + (xi & 0x80) * 7
            xj = xi + 1
            xt1 = xj + (xj & 0x80) * 7
            yt0 = (yi << 7) + ((yi >> 3) << 10)
            yj = yi + 1
            yt1 = (yj << 7) + ((yj >> 3) << 10)
            vz0 = jnp.where(is_yin, 0, VOL3) + (zi << 16)
            zy00 = vz0 + yt0
            zy01 = vz0 + yt1
            zy10 = zy00 + 65536
            zy11 = zy01 + 65536
            idx_v[pl.ds(0 * C + r, 16)] = zy00 + xt0
            idx_v[pl.ds(1 * C + r, 16)] = zy00 + xt1
            idx_v[pl.ds(2 * C + r, 16)] = zy01 + xt0
            idx_v[pl.ds(3 * C + r, 16)] = zy01 + xt1
            idx_v[pl.ds(4 * C + r, 16)] = zy10 + xt0
            idx_v[pl.ds(5 * C + r, 16)] = zy10 + xt1
            idx_v[pl.ds(6 * C + r, 16)] = zy11 + xt0
            idx_v[pl.ds(7 * C + r, 16)] = zy11 + xt1
            return carry2

        lax.fori_loop(0, G, idx_body, 0, unroll=False)

    def fire(b):
        _, idx_v, val_v, _, _, _, _, sem = bufs[b]
        pltpu.async_copy(table_hbm.at[idx_v], val_v, sem)

    def drain(b):
        _, idx_v, val_v, _, _, _, _, sem = bufs[b]
        pltpu.make_async_copy(table_hbm.at[idx_v], val_v, sem).wait()

    def pass2(k, b):
        _, _, val_v, wx_v, wy_v, wz_v, out_v, _ = bufs[b]

        def mix_body(g, carry2):
            r = g * 16
            wx = wx_v[pl.ds(r, 16)]
            wy = wy_v[pl.ds(r, 16)]
            wz = wz_v[pl.ds(r, 16)]
            v000 = val_v[pl.ds(0 * C + r, 16)]
            v001 = val_v[pl.ds(1 * C + r, 16)]
            v010 = val_v[pl.ds(2 * C + r, 16)]
            v011 = val_v[pl.ds(3 * C + r, 16)]
            v100 = val_v[pl.ds(4 * C + r, 16)]
            v101 = val_v[pl.ds(5 * C + r, 16)]
            v110 = val_v[pl.ds(6 * C + r, 16)]
            v111 = val_v[pl.ds(7 * C + r, 16)]
            a00 = v000 + wx * (v001 - v000)
            a01 = v010 + wx * (v011 - v010)
            a10 = v100 + wx * (v101 - v100)
            a11 = v110 + wx * (v111 - v110)
            b0 = a00 + wy * (a01 - a00)
            b1 = a10 + wy * (a11 - a10)
            out_v[pl.ds(r, 16)] = b0 + wz * (b1 - b0)
            return carry2

        lax.fori_loop(0, G, mix_body, 0, unroll=False)
        base = wid * SPW + k * C
        pltpu.sync_copy(out_v, out_hbm.at[pl.ds(base, C)])

    for b in range(DEPTH - 1):
        stage(b, b)
        pass1(b)
        fire(b)

    def pipe_body(k2, carry):
        for b in range(DEPTH):
            k = k2 * DEPTH + b
            kf = k + DEPTH - 1
            bf = (b + DEPTH - 1) % DEPTH

            @pl.when(kf < NCHUNK)
            def _(kf=kf, bf=bf):
                stage(kf, bf)
                pass1(bf)
                fire(bf)

            drain(b)
            pass2(k, b)
        return carry

    lax.fori_loop(0, NCHUNK // DEPTH, pipe_body, 0, unroll=False)


@jax.jit
def _run(samples_t, table):
    mesh = plsc.VectorSubcoreMesh(core_axis_name="c", subcore_axis_name="s")
    f = functools.partial(
        pl.kernel,
        mesh=mesh,
        compiler_params=pltpu.CompilerParams(use_tc_tiling_on_sc=True,
                                             needs_layout_passes=False),
        out_type=jax.ShapeDtypeStruct((N,), jnp.float32),
        scratch_types=[t for _ in range(DEPTH) for t in (
            pltpu.VMEM((7, C), jnp.float32),
            pltpu.VMEM((8 * C,), jnp.int32),
            pltpu.VMEM((8 * C,), jnp.float32),
            pltpu.VMEM((C,), jnp.float32),
            pltpu.VMEM((C,), jnp.float32),
            pltpu.VMEM((C,), jnp.float32),
            pltpu.VMEM((C,), jnp.float32),
            pltpu.SemaphoreType.DMA,
        )],
    )(_tec_body)
    return f(samples_t, table)


def _tile_view(vol):
    v5 = vol.reshape(VOL, VOL // 8, 8, 2, 128)
    return v5.transpose(0, 1, 3, 2, 4).reshape(-1)


def kernel(norm_samples, alpha_volume_yin, alpha_volume_yang):
    table = jnp.concatenate(
        [_tile_view(alpha_volume_yin), _tile_view(alpha_volume_yang)])
    return _run(norm_samples.T, table)
